# baseline (speedup 1.0000x reference)
import jax
import jax.numpy as jnp
import numpy as np
from jax import lax
from jax.experimental import pallas as pl
from jax.experimental.pallas import tpu as pltpu
from jax.experimental.pallas import tpu_sc as plsc

K = 64
N = 8192
NV = N // 16
ROWS = 64
NC = 2
NS = 16
NW = NC * NS
ROWS_PER_W = ROWS // NW
POOL = N + 32

SENT_I = np.int32(-1)
MASK31 = np.int32(0x7FFFFFFF)


def _keyify(v_f32):
    b = lax.bitcast_convert_type(v_f32, jnp.int32)
    m = lax.shift_right_arithmetic(b, 31)
    return lax.bitcast_convert_type(b ^ (~m & MASK31), jnp.uint32)


def _unkeyify(key_u32):
    ki = lax.bitcast_convert_type(key_u32, jnp.int32)
    m = lax.shift_right_arithmetic(ki, 31)
    return lax.bitcast_convert_type(ki ^ (~m & MASK31), jnp.float32)


def _splat_u32(s_i32):
    return jnp.full((16,), s_i32.astype(jnp.uint32), dtype=jnp.uint32)


def _merge2(a, b):
    br = lax.rev(b, (0,))
    return jnp.sort(jnp.minimum(a, br)), jnp.sort(jnp.maximum(a, br))


def _sort64(o0, o1, o2, o3):
    s0, s1, s2, s3 = jnp.sort(o0), jnp.sort(o1), jnp.sort(o2), jnp.sort(o3)
    a0, a1 = _merge2(s0, s1)
    a2, a3 = _merge2(s2, s3)
    r3, r2 = lax.rev(a3, (0,)), lax.rev(a2, (0,))
    l0, l1 = jnp.minimum(a0, r3), jnp.minimum(a1, r2)
    h0, h1 = jnp.maximum(a0, r3), jnp.maximum(a1, r2)
    f0 = jnp.sort(jnp.minimum(l0, l1))
    f1 = jnp.sort(jnp.maximum(l0, l1))
    f2 = jnp.sort(jnp.minimum(h0, h1))
    f3 = jnp.sort(jnp.maximum(h0, h1))
    return f0, f1, f2, f3


def _body(x_hbm, out_hbm, rowbuf, poolA, poolB, hist, hist16,
          counts, outk, outv, sem0):
    wid = lax.axis_index("s") * NC + lax.axis_index("c")
    lanes = lax.iota(jnp.int32, 16)
    laneoff256 = lanes * 256
    laneoff16 = lanes * 16
    zeros16 = jnp.zeros((16,), jnp.int32)
    ones16 = jnp.ones((16,), jnp.int32)
    sentv = jnp.full((16,), SENT_I, jnp.int32)

    @plsc.parallel_loop(0, 256, unroll=4)
    def _(i):
        hist[pl.ds(i * 16, 16)] = zeros16
    for l in range(16):
        hist16[pl.ds(l * 16, 16)] = zeros16

    def compact(src_ref, nb, keep_fn, dst_ref, trash_base, unroll=1):
        @plsc.parallel_loop(0, nb, unroll=unroll, carry=zeros16)
        def offv(i, offv):
            key = lax.bitcast_convert_type(src_ref[pl.ds(i * 16, 16)],
                                           jnp.uint32)
            keep = keep_fn(key)
            km = jnp.where(keep, 1, 0).astype(jnp.int32)
            excl = plsc.cumsum(km) - km
            idx = jnp.where(keep, offv + excl, trash_base + lanes)
            plsc.store_scatter(dst_ref, [idx],
                               lax.bitcast_convert_type(key, jnp.int32))
            pc = plsc.all_reduce_population_count(keep)
            return offv + pc
        n = jnp.max(offv)
        plsc.store_scatter(dst_ref, [jnp.full((16,), n, jnp.int32) + lanes],
                           sentv)
        return n

    row0 = wid * ROWS_PER_W
    cp0 = pltpu.make_async_copy(x_hbm.at[pl.ds(row0, ROWS_PER_W)], rowbuf,
                                sem0)
    cp0.start()
    cp0.wait()

    def row_body(r, _):
        row = row0 + r

        @plsc.parallel_loop(0, NV, unroll=2)
        def _(i):
            key = _keyify(rowbuf[r, pl.ds(i * 16, 16)])
            d = jnp.right_shift(key, np.uint32(24)).astype(jnp.int32)
            plsc.addupdate_scatter(hist, [laneoff256 + d], ones16)

        def sum_iter(j, _):
            acc = zeros16
            for l in range(16):
                acc = acc + hist[pl.ds(l * 256 + j * 16, 16)]
                hist[pl.ds(l * 256 + j * 16, 16)] = zeros16
            counts[pl.ds(j * 16, 16)] = acc
            return 0
        lax.fori_loop(0, 16, sum_iter, 0)

        def scan_iter(j, carry):
            acc, B, G, found = carry
            c = counts[pl.ds(j * 16, 16)]
            tot = jnp.sum(c)
            incl = plsc.cumsum(c)
            maskv = (acc + incl) >= K
            nbef = jnp.sum(jnp.where(maskv, 0, 1))
            excl_at = acc + jnp.sum(jnp.where(lanes == nbef, incl - c, 0))
            hit = jnp.logical_and(acc + tot >= K, jnp.logical_not(found))
            B = jnp.where(hit, j * 16 + nbef, B)
            G = jnp.where(hit, excl_at, G)
            found = jnp.logical_or(found, acc + tot >= K)
            return acc + tot, B, G, found
        _, B0, G0, _ = lax.fori_loop(
            0, 16, scan_iter,
            (jnp.int32(0), jnp.int32(0), jnp.int32(0), jnp.bool_(False)))

        S = G0
        K_need = jnp.int32(K) - S
        Tp = B0

        thr = _unkeyify(_splat_u32(B0 * 16777216 + 16777215))

        @plsc.parallel_loop(0, NV, unroll=2, carry=zeros16)
        def offv(i, offv):
            v = rowbuf[r, pl.ds(i * 16, 16)]
            keep = v >= thr
            km = jnp.where(keep, 1, 0).astype(jnp.int32)
            excl = plsc.cumsum(km) - km
            idx = jnp.where(keep, offv + excl, N + lanes)
            plsc.store_scatter(poolA, [idx],
                               lax.bitcast_convert_type(_keyify(v),
                                                        jnp.int32))
            pc = plsc.all_reduce_population_count(keep)
            return offv + pc
        n = jnp.max(offv)
        plsc.store_scatter(poolA, [jnp.full((16,), n, jnp.int32) + lanes],
                           sentv)

        src, dst = poolA, poolB
        T = None
        Sf = None
        for t in range(1, 7):
            shift = 24 - 4 * t
            pshift = shift + 4
            nb = (n + 15) >> 4

            @plsc.parallel_loop(0, nb)
            def _(i, src=src, shift=shift, pshift=pshift, Tp=Tp):
                key = lax.bitcast_convert_type(src[pl.ds(i * 16, 16)],
                                               jnp.uint32)
                eq = jnp.right_shift(key, np.uint32(pshift)) == _splat_u32(Tp)
                d = (jnp.right_shift(key, np.uint32(shift))
                     & np.uint32(0xF)).astype(jnp.int32)
                plsc.addupdate_scatter(hist16, [laneoff16 + d],
                                       jnp.where(eq, 1, 0).astype(jnp.int32))

            acc = zeros16
            for l in range(16):
                acc = acc + hist16[pl.ds(l * 16, 16)]
                hist16[pl.ds(l * 16, 16)] = zeros16

            incl = plsc.cumsum(acc)
            maskv = incl >= K_need
            B = jnp.sum(jnp.where(maskv, 0, 1))
            G = jnp.sum(jnp.where(lanes == B, incl - acc, 0))
            S = S + G
            K_need = K_need - G
            Tp = Tp * 16 + B

            if t < 6:
                def keep_fn(key, shift=shift, Tp=Tp):
                    return jnp.right_shift(key, np.uint32(shift)) <= \
                        _splat_u32(Tp)
                n = compact(src, nb, keep_fn, dst, N)
                src, dst = dst, src
            else:
                T = Tp

                def keep_lt(key, T=T):
                    return key < _splat_u32(T)
                Sf = compact(src, nb, keep_lt, outk, K)

        Tv = _splat_u32(T)
        os_ = []
        for q in range(4):
            o = lax.bitcast_convert_type(outk[pl.ds(q * 16, 16)], jnp.uint32)
            gidx = lanes + q * 16
            os_.append(jnp.where(gidx < jnp.full((16,), Sf, jnp.int32),
                                 o, Tv))
        f0, f1, f2, f3 = _sort64(*os_)
        outv[0, pl.ds(0, 16)] = _unkeyify(f0)
        outv[0, pl.ds(16, 16)] = _unkeyify(f1)
        outv[0, pl.ds(32, 16)] = _unkeyify(f2)
        outv[0, pl.ds(48, 16)] = _unkeyify(f3)
        pltpu.sync_copy(outv, out_hbm.at[pl.ds(row, 1)])
        return 0

    lax.fori_loop(0, ROWS_PER_W, row_body, 0)


@jax.jit
def kernel(x):
    mesh = plsc.VectorSubcoreMesh(core_axis_name="c", subcore_axis_name="s",
                                  num_cores=NC, num_subcores=NS)
    run = pl.kernel(
        _body,
        out_type=jax.ShapeDtypeStruct((ROWS, K), jnp.float32),
        mesh=mesh,
        compiler_params=pltpu.CompilerParams(
            needs_layout_passes=False,
            disable_bounds_checks=True,
            disable_semaphore_checks=True,
            skip_device_barrier=True,
        ),
        scratch_types=[
            pltpu.VMEM((ROWS_PER_W, N), jnp.float32),
            pltpu.VMEM((POOL,), jnp.int32),
            pltpu.VMEM((POOL,), jnp.int32),
            pltpu.VMEM((4096,), jnp.int32),
            pltpu.VMEM((256,), jnp.int32),
            pltpu.VMEM((256,), jnp.int32),
            pltpu.VMEM((K + 32,), jnp.int32),
            pltpu.VMEM((1, K), jnp.float32),
            pltpu.SemaphoreType.DMA,
        ],
    )
    return run(x)


# speedup vs baseline: 1.0307x; 1.0307x over previous
import jax
import jax.numpy as jnp
import numpy as np
from jax import lax
from jax.experimental import pallas as pl
from jax.experimental.pallas import tpu as pltpu
from jax.experimental.pallas import tpu_sc as plsc

K = 64
N = 8192
NV = N // 16
ROWS = 64
NC = 2
NS = 16
NW = NC * NS
ROWS_PER_W = ROWS // NW
POOL = N + 32

SENT_I = np.int32(-1)
MASK31 = np.int32(0x7FFFFFFF)


def _keyify(v_f32):
    b = lax.bitcast_convert_type(v_f32, jnp.int32)
    m = lax.shift_right_arithmetic(b, 31)
    return lax.bitcast_convert_type(b ^ (~m & MASK31), jnp.uint32)


def _unkeyify(key_u32):
    ki = lax.bitcast_convert_type(key_u32, jnp.int32)
    m = lax.shift_right_arithmetic(ki, 31)
    return lax.bitcast_convert_type(ki ^ (~m & MASK31), jnp.float32)


def _splat_u32(s_i32):
    return jnp.full((16,), s_i32.astype(jnp.uint32), dtype=jnp.uint32)


def _merge2(a, b):
    br = lax.rev(b, (0,))
    return jnp.sort(jnp.minimum(a, br)), jnp.sort(jnp.maximum(a, br))


def _sort64(o0, o1, o2, o3):
    s0, s1, s2, s3 = jnp.sort(o0), jnp.sort(o1), jnp.sort(o2), jnp.sort(o3)
    a0, a1 = _merge2(s0, s1)
    a2, a3 = _merge2(s2, s3)
    r3, r2 = lax.rev(a3, (0,)), lax.rev(a2, (0,))
    l0, l1 = jnp.minimum(a0, r3), jnp.minimum(a1, r2)
    h0, h1 = jnp.maximum(a0, r3), jnp.maximum(a1, r2)
    f0 = jnp.sort(jnp.minimum(l0, l1))
    f1 = jnp.sort(jnp.maximum(l0, l1))
    f2 = jnp.sort(jnp.minimum(h0, h1))
    f3 = jnp.sort(jnp.maximum(h0, h1))
    return f0, f1, f2, f3


def _body(x_hbm, out_hbm, rowbuf, poolA, poolB, hist, hist16,
          counts, outk, outv, sem0, sem1):
    wid = lax.axis_index("s") * NC + lax.axis_index("c")
    lanes = lax.iota(jnp.int32, 16)
    laneoff256 = lanes * 256
    laneoff16 = lanes * 16
    zeros16 = jnp.zeros((16,), jnp.int32)
    ones16 = jnp.ones((16,), jnp.int32)
    sentv = jnp.full((16,), SENT_I, jnp.int32)

    @plsc.parallel_loop(0, 256, unroll=8)
    def _(i):
        hist[pl.ds(i * 16, 16)] = zeros16
    for l in range(16):
        hist16[pl.ds(l * 16, 16)] = zeros16

    def compact(src_ref, nb, keep_fn, dst_ref, trash_base, unroll=1):
        @plsc.parallel_loop(0, nb, unroll=unroll, carry=zeros16)
        def offv(i, offv):
            key = lax.bitcast_convert_type(src_ref[pl.ds(i * 16, 16)],
                                           jnp.uint32)
            keep = keep_fn(key)
            km = jnp.where(keep, 1, 0).astype(jnp.int32)
            excl = plsc.cumsum(km) - km
            idx = jnp.where(keep, offv + excl, trash_base + lanes)
            plsc.store_scatter(dst_ref, [idx],
                               lax.bitcast_convert_type(key, jnp.int32))
            pc = plsc.all_reduce_population_count(keep)
            return offv + pc
        n = jnp.max(offv)
        plsc.store_scatter(dst_ref, [jnp.full((16,), n, jnp.int32) + lanes],
                           sentv)
        return n

    row0 = wid * ROWS_PER_W
    cp0 = pltpu.make_async_copy(x_hbm.at[pl.ds(row0, 1)],
                                rowbuf.at[pl.ds(0, 1)], sem0)
    cp0.start()
    cp1 = pltpu.make_async_copy(x_hbm.at[pl.ds(row0 + 1, 1)],
                                rowbuf.at[pl.ds(1, 1)], sem1)
    cp1.start()
    cp0.wait()

    def row_body(r, _):
        row = row0 + r

        @pl.when(r == 1)
        def _():
            cp1.wait()

        @plsc.parallel_loop(0, NV, unroll=4)
        def _(i):
            key = _keyify(rowbuf[r, pl.ds(i * 16, 16)])
            d = jnp.right_shift(key, np.uint32(24)).astype(jnp.int32)
            plsc.addupdate_scatter(hist, [laneoff256 + d], ones16)

        def sum_iter(j, _):
            acc = zeros16
            for l in range(16):
                acc = acc + hist[pl.ds(l * 256 + j * 16, 16)]
                hist[pl.ds(l * 256 + j * 16, 16)] = zeros16
            counts[pl.ds(j * 16, 16)] = acc
            return 0
        lax.fori_loop(0, 16, sum_iter, 0)

        def scan_iter(j, carry):
            acc, B, G, found = carry
            c = counts[pl.ds(j * 16, 16)]
            tot = jnp.sum(c)
            incl = plsc.cumsum(c)
            maskv = (acc + incl) >= K
            nbef = jnp.sum(jnp.where(maskv, 0, 1))
            excl_at = acc + jnp.sum(jnp.where(lanes == nbef, incl - c, 0))
            hit = jnp.logical_and(acc + tot >= K, jnp.logical_not(found))
            B = jnp.where(hit, j * 16 + nbef, B)
            G = jnp.where(hit, excl_at, G)
            found = jnp.logical_or(found, acc + tot >= K)
            return acc + tot, B, G, found
        _, B0, G0, _ = lax.fori_loop(
            0, 16, scan_iter,
            (jnp.int32(0), jnp.int32(0), jnp.int32(0), jnp.bool_(False)))

        S = G0
        K_need = jnp.int32(K) - S
        Tp = B0

        thr = _unkeyify(_splat_u32(B0 * 16777216 + 16777215))

        @plsc.parallel_loop(0, NV, unroll=4, carry=zeros16)
        def offv(i, offv):
            v = rowbuf[r, pl.ds(i * 16, 16)]
            keep = v >= thr
            km = jnp.where(keep, 1, 0).astype(jnp.int32)
            excl = plsc.cumsum(km) - km
            idx = jnp.where(keep, offv + excl, N + lanes)
            plsc.store_scatter(poolA, [idx],
                               lax.bitcast_convert_type(_keyify(v),
                                                        jnp.int32))
            pc = plsc.all_reduce_population_count(keep)
            return offv + pc
        n = jnp.max(offv)
        plsc.store_scatter(poolA, [jnp.full((16,), n, jnp.int32) + lanes],
                           sentv)

        src, dst = poolA, poolB
        T = None
        Sf = None
        for t in range(1, 7):
            shift = 24 - 4 * t
            pshift = shift + 4
            nb = (n + 15) >> 4

            @plsc.parallel_loop(0, nb)
            def _(i, src=src, shift=shift, pshift=pshift, Tp=Tp):
                key = lax.bitcast_convert_type(src[pl.ds(i * 16, 16)],
                                               jnp.uint32)
                eq = jnp.right_shift(key, np.uint32(pshift)) == _splat_u32(Tp)
                d = (jnp.right_shift(key, np.uint32(shift))
                     & np.uint32(0xF)).astype(jnp.int32)
                plsc.addupdate_scatter(hist16, [laneoff16 + d],
                                       jnp.where(eq, 1, 0).astype(jnp.int32))

            acc = zeros16
            for l in range(16):
                acc = acc + hist16[pl.ds(l * 16, 16)]
                hist16[pl.ds(l * 16, 16)] = zeros16

            incl = plsc.cumsum(acc)
            maskv = incl >= K_need
            B = jnp.sum(jnp.where(maskv, 0, 1))
            G = jnp.sum(jnp.where(lanes == B, incl - acc, 0))
            S = S + G
            K_need = K_need - G
            Tp = Tp * 16 + B

            if t < 6:
                def keep_fn(key, shift=shift, Tp=Tp):
                    return jnp.right_shift(key, np.uint32(shift)) <= \
                        _splat_u32(Tp)
                n = compact(src, nb, keep_fn, dst, N)
                src, dst = dst, src
            else:
                T = Tp

                def keep_lt(key, T=T):
                    return key < _splat_u32(T)
                Sf = compact(src, nb, keep_lt, outk, K)

        Tv = _splat_u32(T)
        os_ = []
        for q in range(4):
            o = lax.bitcast_convert_type(outk[pl.ds(q * 16, 16)], jnp.uint32)
            gidx = lanes + q * 16
            os_.append(jnp.where(gidx < jnp.full((16,), Sf, jnp.int32),
                                 o, Tv))
        f0, f1, f2, f3 = _sort64(*os_)
        outv[0, pl.ds(0, 16)] = _unkeyify(f0)
        outv[0, pl.ds(16, 16)] = _unkeyify(f1)
        outv[0, pl.ds(32, 16)] = _unkeyify(f2)
        outv[0, pl.ds(48, 16)] = _unkeyify(f3)
        pltpu.sync_copy(outv, out_hbm.at[pl.ds(row, 1)])
        return 0

    lax.fori_loop(0, ROWS_PER_W, row_body, 0)


@jax.jit
def kernel(x):
    mesh = plsc.VectorSubcoreMesh(core_axis_name="c", subcore_axis_name="s",
                                  num_cores=NC, num_subcores=NS)
    run = pl.kernel(
        _body,
        out_type=jax.ShapeDtypeStruct((ROWS, K), jnp.float32),
        mesh=mesh,
        compiler_params=pltpu.CompilerParams(
            needs_layout_passes=False,
            disable_bounds_checks=True,
            disable_semaphore_checks=True,
            skip_device_barrier=True,
        ),
        scratch_types=[
            pltpu.VMEM((ROWS_PER_W, N), jnp.float32),
            pltpu.VMEM((POOL,), jnp.int32),
            pltpu.VMEM((POOL,), jnp.int32),
            pltpu.VMEM((4096,), jnp.int32),
            pltpu.VMEM((256,), jnp.int32),
            pltpu.VMEM((256,), jnp.int32),
            pltpu.VMEM((K + 32,), jnp.int32),
            pltpu.VMEM((1, K), jnp.float32),
            pltpu.SemaphoreType.DMA,
            pltpu.SemaphoreType.DMA,
        ],
    )
    return run(x)
